# Initial kernel scaffold; baseline (speedup 1.0000x reference)
import jax
import jax.numpy as jnp
from jax.experimental import pallas as pl


def kernel(obs_emb, observed_mask, lengths, avg_interval, var_plm_rep, rarity_W, Wf1, bf1, Wf2, bf2, Wg1, bg1, Wg2, bg2, Wu, bu, Wr, br, Wc, bc):
    raise NotImplementedError("write your pallas kernel here")


# speedup vs baseline: 1.4859x; 1.4859x over previous
import jax
import jax.numpy as jnp
from jax.experimental import pallas as pl
from jax.experimental.pallas import tpu as pltpu

_BATCH, _STEPS, _NODES = 64, 24, 64
_D, _QDIM, _PLM = 32, 5, 768
_ALPHA = 0.5
_BB = 8
_INF = 2 * _D + 1
_R = _BB * _NODES
_PREC = jax.lax.Precision.HIGHEST


def _rnn_body(obs_ref, mask_ref, maskT_ref, avg_ref, avgT_ref, len_ref,
              vpr_ref, rW_ref, Wf1_ref, bf1_ref, Wf2_ref, bf2_ref,
              Wg1_ref, bg1_ref, Wg2_ref, bg2_ref,
              Wru_ref, bru_ref, Wc_ref, bc_ref,
              out_ref,
              adj_s, qvb_s, bbru_s, bbc_s):

    @pl.when(pl.program_id(0) == 0)
    def _prologue():
        vpr = vpr_ref[...]
        qh = jnp.maximum(
            jax.lax.dot(vpr, Wf1_ref[...], precision=_PREC) + bf1_ref[...], 0.0)
        qv = jax.lax.dot(qh, Wf2_ref[...], precision=_PREC) + bf2_ref[...]
        gh = jnp.maximum(
            jax.lax.dot(vpr, Wg1_ref[...], precision=_PREC) + bg1_ref[...], 0.0)
        ne = jax.lax.dot(gh, Wg2_ref[...], precision=_PREC) + bg2_ref[...]
        nrm = jnp.sqrt(jnp.sum(ne * ne, axis=1, keepdims=True))
        ne = ne / jnp.maximum(nrm, 1e-12)
        logits = jax.lax.dot_general(ne, ne, (((1,), (1,)), ((), ())),
                                     precision=_PREC)
        mx = jnp.max(logits, axis=1, keepdims=True)
        e = jnp.exp(logits - mx)
        adj_s[...] = e / jnp.sum(e, axis=1, keepdims=True)
        qblk = jnp.concatenate(
            [jnp.broadcast_to(qv[:, d:d + 1], (_NODES, _INF))
             for d in range(_QDIM)], axis=1)
        qvb_s[...] = jnp.concatenate([qblk] * _BB, axis=0)
        bbru = jax.lax.dot(qv, bru_ref[...], precision=_PREC)
        bbru_s[...] = jnp.concatenate([bbru] * _BB, axis=0)
        bbc = jax.lax.dot(qv, bc_ref[...], precision=_PREC)
        bbc_s[...] = jnp.concatenate([bbc] * _BB, axis=0)

    vto = jnp.sum(mask_ref[...], axis=1)
    vtoT = jnp.sum(maskT_ref[0], axis=0)
    lb = len_ref[...]
    ls = jnp.concatenate(
        [jnp.broadcast_to(lb[b:b + 1, :], (_NODES, 1)) for b in range(_BB)],
        axis=0)
    eye = (jax.lax.broadcasted_iota(jnp.int32, (_NODES, _NODES), 0) ==
           jax.lax.broadcasted_iota(jnp.int32, (_NODES, _NODES), 1)
           ).astype(jnp.float32)
    adj = adj_s[...]
    qvb = qvb_s[...]
    bbru = bbru_s[...]
    bbc = bbc_s[...]
    rW = rW_ref[...]
    Wru = Wru_ref[...]
    Wc = Wc_ref[...]

    def step_fn(step, carry):
        h, out = carry
        m_lane = mask_ref[:, step, :]
        mT = maskT_ref[0, step]
        a_lane = avg_ref[:, step, :]
        aT = avgT_ref[0, step]
        rar_lane = _ALPHA * jnp.tanh(a_lane / (vto + 1.0))
        rarT = _ALPHA * jnp.tanh(aT / (vtoT + 1.0))
        rar_sl = [rarT[:, b:b + 1] for b in range(_BB)]
        m_sl = [mT[:, b:b + 1] for b in range(_BB)]
        rar_col = jnp.concatenate(rar_sl, axis=0)
        m_col = jnp.concatenate(m_sl, axis=0)
        rar_rows = jnp.stack(rar_sl, axis=0)
        m_rows = jnp.stack(m_sl, axis=0)
        rsm = -rW[None] * jnp.abs(rar_rows - rar_lane[:, None, :])
        amask = m_rows * m_lane[:, None, :]
        cadj = (adj[None] * (1.0 + rsm) * amask * (1.0 - eye[None])
                + eye[None])
        obs = obs_ref[:, step].reshape(_R, _D)
        xh = jnp.concatenate([obs, rar_col, h], axis=1)
        comb = jnp.concatenate(
            [jax.lax.dot(cadj[b], xh[b * _NODES:(b + 1) * _NODES, :],
                         precision=_PREC) for b in range(_BB)], axis=0)
        comb5 = jnp.concatenate([comb] * _QDIM, axis=1) * qvb
        acc = jax.lax.dot(comb5, Wru, precision=_PREC) + bbru
        r = jax.nn.sigmoid(acc[:, :_D])
        u = jax.nn.sigmoid(acc[:, _D:2 * _D])
        mgt = m_col > 0.0
        h_r = jnp.where(mgt, r * h, h)
        xc = jnp.concatenate([obs, rar_col, h_r], axis=1)
        xc5 = jnp.concatenate([xc] * _QDIM, axis=1) * qvb
        cand = jnp.tanh(jax.lax.dot(xc5, Wc, precision=_PREC) + bbc)
        h_new = jnp.where(mgt, (1.0 - u) * h_r + u * cand, h)
        out_new = jnp.where(ls == step + 1, h_new, out)
        return h_new, out_new

    h0 = jnp.zeros((_R, _D), jnp.float32)
    _, out = jax.lax.fori_loop(0, _STEPS, step_fn, (h0, h0))
    out_ref[...] = out.reshape(_BB, _NODES, _D)


def kernel(obs_emb, observed_mask, lengths, avg_interval, var_plm_rep,
           rarity_W, Wf1, bf1, Wf2, bf2, Wg1, bg1, Wg2, bg2,
           Wu, bu, Wr, br, Wc, bc):
    maskT = (observed_mask.transpose(1, 2, 0)
             .reshape(_STEPS, _NODES, _BATCH // _BB, _BB)
             .transpose(2, 0, 1, 3))
    avgT = (avg_interval.transpose(1, 2, 0)
            .reshape(_STEPS, _NODES, _BATCH // _BB, _BB)
            .transpose(2, 0, 1, 3))
    Wru = jnp.stack([Wr, Wu], axis=2).reshape(_QDIM * _INF, 2 * _D)
    bru = jnp.concatenate([br, bu], axis=1)
    Wcf = Wc.reshape(_QDIM * _INF, _D)

    full = lambda nd: (lambda i: (0,) * nd)
    out = pl.pallas_call(
        _rnn_body,
        grid=(_BATCH // _BB,),
        in_specs=[
            pl.BlockSpec((_BB, _STEPS, _NODES, _D), lambda i: (i, 0, 0, 0)),
            pl.BlockSpec((_BB, _STEPS, _NODES), lambda i: (i, 0, 0)),
            pl.BlockSpec((1, _STEPS, _NODES, _BB), lambda i: (i, 0, 0, 0)),
            pl.BlockSpec((_BB, _STEPS, _NODES), lambda i: (i, 0, 0)),
            pl.BlockSpec((1, _STEPS, _NODES, _BB), lambda i: (i, 0, 0, 0)),
            pl.BlockSpec((_BB, 1), lambda i: (i, 0)),
            pl.BlockSpec((_NODES, _PLM), full(2)),
            pl.BlockSpec((_NODES, _NODES), full(2)),
            pl.BlockSpec((_PLM, 2 * _D), full(2)),
            pl.BlockSpec((1, 2 * _D), full(2)),
            pl.BlockSpec((2 * _D, _QDIM), full(2)),
            pl.BlockSpec((1, _QDIM), full(2)),
            pl.BlockSpec((_PLM, 2 * _D), full(2)),
            pl.BlockSpec((1, 2 * _D), full(2)),
            pl.BlockSpec((2 * _D, 8), full(2)),
            pl.BlockSpec((1, 8), full(2)),
            pl.BlockSpec((_QDIM * _INF, 2 * _D), full(2)),
            pl.BlockSpec((_QDIM, 2 * _D), full(2)),
            pl.BlockSpec((_QDIM * _INF, _D), full(2)),
            pl.BlockSpec((_QDIM, _D), full(2)),
        ],
        out_specs=pl.BlockSpec((_BB, _NODES, _D), lambda i: (i, 0, 0)),
        out_shape=jax.ShapeDtypeStruct((_BATCH, _NODES, _D), jnp.float32),
        scratch_shapes=[
            pltpu.VMEM((_NODES, _NODES), jnp.float32),
            pltpu.VMEM((_R, _QDIM * _INF), jnp.float32),
            pltpu.VMEM((_R, 2 * _D), jnp.float32),
            pltpu.VMEM((_R, _D), jnp.float32),
        ],
        compiler_params=pltpu.CompilerParams(
            dimension_semantics=("arbitrary",)),
    )(obs_emb, observed_mask, maskT, avg_interval, avgT, lengths,
      var_plm_rep, rarity_W, Wf1, bf1.reshape(1, -1), Wf2, bf2.reshape(1, -1),
      Wg1, bg1.reshape(1, -1), Wg2, bg2.reshape(1, -1),
      Wru, bru, Wcf, bc)
    return out


# speedup vs baseline: 2.3279x; 1.5667x over previous
import jax
import jax.numpy as jnp
from jax.experimental import pallas as pl
from jax.experimental.pallas import tpu as pltpu

_BATCH, _STEPS, _NODES = 64, 24, 64
_D, _QDIM, _PLM = 32, 5, 768
_ALPHA = 0.5
_BB = 8
_INF = 2 * _D + 1
_R = _BB * _NODES
_PREC = jax.lax.Precision.DEFAULT


def _rnn_body(obs_ref, mask_ref, maskT_ref, avg_ref, avgT_ref, len_ref,
              vpr_ref, rW_ref, Wf1_ref, bf1_ref, Wf2_ref, bf2_ref,
              Wg1_ref, bg1_ref, Wg2_ref, bg2_ref,
              Wru_ref, bru_ref, Wc_ref, bc_ref,
              out_ref,
              adj_s, qvb_s, bbru_s, bbc_s):

    @pl.when(pl.program_id(0) == 0)
    def _prologue():
        vpr = vpr_ref[...]
        qh = jnp.maximum(
            jax.lax.dot(vpr, Wf1_ref[...], precision=_PREC) + bf1_ref[...], 0.0)
        qv = jax.lax.dot(qh, Wf2_ref[...], precision=_PREC) + bf2_ref[...]
        gh = jnp.maximum(
            jax.lax.dot(vpr, Wg1_ref[...], precision=_PREC) + bg1_ref[...], 0.0)
        ne = jax.lax.dot(gh, Wg2_ref[...], precision=_PREC) + bg2_ref[...]
        nrm = jnp.sqrt(jnp.sum(ne * ne, axis=1, keepdims=True))
        ne = ne / jnp.maximum(nrm, 1e-12)
        logits = jax.lax.dot_general(ne, ne, (((1,), (1,)), ((), ())),
                                     precision=_PREC)
        mx = jnp.max(logits, axis=1, keepdims=True)
        e = jnp.exp(logits - mx)
        adj_s[...] = e / jnp.sum(e, axis=1, keepdims=True)
        qblk = jnp.concatenate(
            [jnp.broadcast_to(qv[:, d:d + 1], (_NODES, _INF))
             for d in range(_QDIM)], axis=1)
        qvb_s[...] = jnp.concatenate([qblk] * _BB, axis=0)
        bbru = jax.lax.dot(qv, bru_ref[...], precision=_PREC)
        bbru_s[...] = jnp.concatenate([bbru] * _BB, axis=0)
        bbc = jax.lax.dot(qv, bc_ref[...], precision=_PREC)
        bbc_s[...] = jnp.concatenate([bbc] * _BB, axis=0)

    vto = jnp.sum(mask_ref[...], axis=1)
    vtoT = jnp.sum(maskT_ref[0], axis=0)
    lb = len_ref[...]
    ls = jnp.concatenate(
        [jnp.broadcast_to(lb[b:b + 1, :], (_NODES, 1)) for b in range(_BB)],
        axis=0)
    eye = (jax.lax.broadcasted_iota(jnp.int32, (_NODES, _NODES), 0) ==
           jax.lax.broadcasted_iota(jnp.int32, (_NODES, _NODES), 1)
           ).astype(jnp.float32)
    adj = adj_s[...]
    qvb = qvb_s[...]
    bbru = bbru_s[...]
    bbc = bbc_s[...]
    rW = rW_ref[...]
    Wru = Wru_ref[...]
    Wc = Wc_ref[...]

    def step_fn(step, carry):
        h, out = carry
        m_lane = mask_ref[:, step, :]
        mT = maskT_ref[0, step]
        a_lane = avg_ref[:, step, :]
        aT = avgT_ref[0, step]
        rar_lane = _ALPHA * jnp.tanh(a_lane / (vto + 1.0))
        rarT = _ALPHA * jnp.tanh(aT / (vtoT + 1.0))
        rar_sl = [rarT[:, b:b + 1] for b in range(_BB)]
        m_sl = [mT[:, b:b + 1] for b in range(_BB)]
        rar_col = jnp.concatenate(rar_sl, axis=0)
        m_col = jnp.concatenate(m_sl, axis=0)
        rar_rows = jnp.stack(rar_sl, axis=0)
        m_rows = jnp.stack(m_sl, axis=0)
        rsm = -rW[None] * jnp.abs(rar_rows - rar_lane[:, None, :])
        amask = m_rows * m_lane[:, None, :]
        cadj = (adj[None] * (1.0 + rsm) * amask * (1.0 - eye[None])
                + eye[None])
        obs = obs_ref[:, step].reshape(_R, _D)
        xh = jnp.concatenate([obs, rar_col, h], axis=1)
        comb = jnp.concatenate(
            [jax.lax.dot(cadj[b], xh[b * _NODES:(b + 1) * _NODES, :],
                         precision=_PREC) for b in range(_BB)], axis=0)
        comb5 = jnp.concatenate([comb] * _QDIM, axis=1) * qvb
        acc = jax.lax.dot(comb5, Wru, precision=_PREC) + bbru
        r = jax.nn.sigmoid(acc[:, :_D])
        u = jax.nn.sigmoid(acc[:, _D:2 * _D])
        mgt = m_col > 0.0
        h_r = jnp.where(mgt, r * h, h)
        xc = jnp.concatenate([obs, rar_col, h_r], axis=1)
        xc5 = jnp.concatenate([xc] * _QDIM, axis=1) * qvb
        cand = jnp.tanh(jax.lax.dot(xc5, Wc, precision=_PREC) + bbc)
        h_new = jnp.where(mgt, (1.0 - u) * h_r + u * cand, h)
        out_new = jnp.where(ls == step + 1, h_new, out)
        return h_new, out_new

    h0 = jnp.zeros((_R, _D), jnp.float32)
    _, out = jax.lax.fori_loop(0, _STEPS, step_fn, (h0, h0))
    out_ref[...] = out.reshape(_BB, _NODES, _D)


def kernel(obs_emb, observed_mask, lengths, avg_interval, var_plm_rep,
           rarity_W, Wf1, bf1, Wf2, bf2, Wg1, bg1, Wg2, bg2,
           Wu, bu, Wr, br, Wc, bc):
    maskT = (observed_mask.transpose(1, 2, 0)
             .reshape(_STEPS, _NODES, _BATCH // _BB, _BB)
             .transpose(2, 0, 1, 3))
    avgT = (avg_interval.transpose(1, 2, 0)
            .reshape(_STEPS, _NODES, _BATCH // _BB, _BB)
            .transpose(2, 0, 1, 3))
    Wru = jnp.stack([Wr, Wu], axis=2).reshape(_QDIM * _INF, 2 * _D)
    bru = jnp.concatenate([br, bu], axis=1)
    Wcf = Wc.reshape(_QDIM * _INF, _D)

    full = lambda nd: (lambda i: (0,) * nd)
    out = pl.pallas_call(
        _rnn_body,
        grid=(_BATCH // _BB,),
        in_specs=[
            pl.BlockSpec((_BB, _STEPS, _NODES, _D), lambda i: (i, 0, 0, 0)),
            pl.BlockSpec((_BB, _STEPS, _NODES), lambda i: (i, 0, 0)),
            pl.BlockSpec((1, _STEPS, _NODES, _BB), lambda i: (i, 0, 0, 0)),
            pl.BlockSpec((_BB, _STEPS, _NODES), lambda i: (i, 0, 0)),
            pl.BlockSpec((1, _STEPS, _NODES, _BB), lambda i: (i, 0, 0, 0)),
            pl.BlockSpec((_BB, 1), lambda i: (i, 0)),
            pl.BlockSpec((_NODES, _PLM), full(2)),
            pl.BlockSpec((_NODES, _NODES), full(2)),
            pl.BlockSpec((_PLM, 2 * _D), full(2)),
            pl.BlockSpec((1, 2 * _D), full(2)),
            pl.BlockSpec((2 * _D, _QDIM), full(2)),
            pl.BlockSpec((1, _QDIM), full(2)),
            pl.BlockSpec((_PLM, 2 * _D), full(2)),
            pl.BlockSpec((1, 2 * _D), full(2)),
            pl.BlockSpec((2 * _D, 8), full(2)),
            pl.BlockSpec((1, 8), full(2)),
            pl.BlockSpec((_QDIM * _INF, 2 * _D), full(2)),
            pl.BlockSpec((_QDIM, 2 * _D), full(2)),
            pl.BlockSpec((_QDIM * _INF, _D), full(2)),
            pl.BlockSpec((_QDIM, _D), full(2)),
        ],
        out_specs=pl.BlockSpec((_BB, _NODES, _D), lambda i: (i, 0, 0)),
        out_shape=jax.ShapeDtypeStruct((_BATCH, _NODES, _D), jnp.float32),
        scratch_shapes=[
            pltpu.VMEM((_NODES, _NODES), jnp.float32),
            pltpu.VMEM((_R, _QDIM * _INF), jnp.float32),
            pltpu.VMEM((_R, 2 * _D), jnp.float32),
            pltpu.VMEM((_R, _D), jnp.float32),
        ],
        compiler_params=pltpu.CompilerParams(
            dimension_semantics=("arbitrary",)),
    )(obs_emb, observed_mask, maskT, avg_interval, avgT, lengths,
      var_plm_rep, rarity_W, Wf1, bf1.reshape(1, -1), Wf2, bf2.reshape(1, -1),
      Wg1, bg1.reshape(1, -1), Wg2, bg2.reshape(1, -1),
      Wru, bru, Wcf, bc)
    return out
